# Initial kernel scaffold; baseline (speedup 1.0000x reference)
import functools

import jax
import jax.numpy as jnp
from jax import lax
from jax.experimental import pallas as pl
from jax.experimental.pallas import tpu as pltpu
from jax.experimental.pallas import tpu_sc as plsc

VOCAB = 1000000
EMBED = 64
HIDDEN = 256
OUT = 1
BATCH = 16384
HIST = 200

NC = 2
NS = 16
NW = NC * NS
ROWS_PER_W = BATCH // NW
CHUNK = 8
G0 = 128
G1 = HIST - G0


def _sc_pool(idx_flat, table):
    mesh = plsc.VectorSubcoreMesh(core_axis_name="c", subcore_axis_name="s")

    @functools.partial(
        pl.kernel,
        mesh=mesh,
        out_type=jax.ShapeDtypeStruct((BATCH, EMBED), jnp.float32),
        scratch_types=[
            pltpu.VMEM((CHUNK * HIST,), jnp.int32),
            pltpu.VMEM((HIST, EMBED), jnp.float32),
            pltpu.VMEM((CHUNK, EMBED), jnp.float32),
            pltpu.SemaphoreType.DMA,
        ],
    )
    def pool(idx_hbm, table_hbm, out_hbm, idx_v, rows_v, acc_v, sem):
        wid = lax.axis_index("s") * NC + lax.axis_index("c")
        base = wid * ROWS_PER_W

        def chunk_body(ci, _):
            row0 = base + ci * CHUNK
            pltpu.sync_copy(idx_hbm.at[pl.ds(row0 * HIST, CHUNK * HIST)], idx_v)

            def row_body(r, _):
                cp0 = pltpu.async_copy(
                    table_hbm.at[idx_v.at[pl.ds(r * HIST, G0)]],
                    rows_v.at[pl.ds(0, G0)], sem)
                cp1 = pltpu.async_copy(
                    table_hbm.at[idx_v.at[pl.ds(r * HIST + G0, G1)]],
                    rows_v.at[pl.ds(G0, G1)], sem)
                cp0.wait()
                cp1.wait()

                def acc_body(j, carry):
                    a0, a1, a2, a3 = carry
                    a0 = a0 + rows_v[j, pl.ds(0, 16)]
                    a1 = a1 + rows_v[j, pl.ds(16, 16)]
                    a2 = a2 + rows_v[j, pl.ds(32, 16)]
                    a3 = a3 + rows_v[j, pl.ds(48, 16)]
                    return a0, a1, a2, a3

                z = jnp.zeros((16,), jnp.float32)
                a0, a1, a2, a3 = lax.fori_loop(0, HIST, acc_body, (z, z, z, z))
                acc_v[r, pl.ds(0, 16)] = a0
                acc_v[r, pl.ds(16, 16)] = a1
                acc_v[r, pl.ds(32, 16)] = a2
                acc_v[r, pl.ds(48, 16)] = a3
                return 0

            lax.fori_loop(0, CHUNK, row_body, 0)
            pltpu.sync_copy(acc_v, out_hbm.at[pl.ds(row0, CHUNK)])
            return 0

        lax.fori_loop(0, ROWS_PER_W // CHUNK, chunk_body, 0)

    return pool(idx_flat, table)


def _tc_mlp(x, w1, b1, w2, b2):
    TB = 2048

    def body(x_ref, w1_ref, b1_ref, w2_ref, b2_ref, o_ref):
        h = jnp.dot(x_ref[...], w1_ref[...],
                    preferred_element_type=jnp.float32) + b1_ref[...]
        h = h * (1.0 / (1.0 + jnp.exp(-h)))
        o = jnp.dot(h, w2_ref[...], preferred_element_type=jnp.float32) + b2_ref[...]
        o_ref[...] = 1.0 / (1.0 + jnp.exp(-o))

    return pl.pallas_call(
        body,
        grid=(BATCH // TB,),
        in_specs=[
            pl.BlockSpec((TB, EMBED), lambda i: (i, 0)),
            pl.BlockSpec((EMBED, HIDDEN), lambda i: (0, 0)),
            pl.BlockSpec((1, HIDDEN), lambda i: (0, 0)),
            pl.BlockSpec((HIDDEN, OUT), lambda i: (0, 0)),
            pl.BlockSpec((1, OUT), lambda i: (0, 0)),
        ],
        out_specs=pl.BlockSpec((TB, OUT), lambda i: (i, 0)),
        out_shape=jax.ShapeDtypeStruct((BATCH, OUT), jnp.float32),
    )(x, w1, b1, w2, b2)


def kernel(indices, table, W1, b1, W2, b2):
    idx_flat = jnp.reshape(indices, (-1,))
    sums = _sc_pool(idx_flat, table)
    w1s = W1 * (1.0 / HIST)
    return _tc_mlp(sums, w1s, jnp.reshape(b1, (1, HIDDEN)),
                   W2, jnp.reshape(b2, (1, OUT)))


# speedup vs baseline: 1.9605x; 1.9605x over previous
import functools

import jax
import jax.numpy as jnp
from jax import lax
from jax.experimental import pallas as pl
from jax.experimental.pallas import tpu as pltpu
from jax.experimental.pallas import tpu_sc as plsc

VOCAB = 1000000
EMBED = 64
HIDDEN = 256
OUT = 1
BATCH = 16384
HIST = 200

NC = 2
NS = 16
NW = NC * NS
ROWS_PER_W = BATCH // NW
CHUNK = 8
G0 = 128
G1 = HIST - G0


def _sc_pool(idx_flat, table):
    mesh = plsc.VectorSubcoreMesh(core_axis_name="c", subcore_axis_name="s")

    @functools.partial(
        pl.kernel,
        mesh=mesh,
        out_type=jax.ShapeDtypeStruct((BATCH, EMBED), jnp.float32),
        scratch_types=[
            pltpu.VMEM((CHUNK * HIST,), jnp.int32),
            pltpu.VMEM((HIST, EMBED), jnp.float32),
            pltpu.VMEM((CHUNK, EMBED), jnp.float32),
            pltpu.SemaphoreType.DMA,
        ],
        compiler_params=pltpu.CompilerParams(use_tc_tiling_on_sc=False),
    )
    def pool(idx_hbm, table_hbm, out_hbm, idx_v, rows_v, acc_v, sem):
        wid = lax.axis_index("s") * NC + lax.axis_index("c")
        base = wid * ROWS_PER_W

        def chunk_body(ci, _):
            row0 = base + ci * CHUNK
            pltpu.sync_copy(idx_hbm.at[pl.ds(row0 * HIST, CHUNK * HIST)], idx_v)

            def row_body(r, _):
                cp0 = pltpu.async_copy(
                    table_hbm.at[idx_v.at[pl.ds(r * HIST, G0)]],
                    rows_v.at[pl.ds(0, G0)], sem)
                cp1 = pltpu.async_copy(
                    table_hbm.at[idx_v.at[pl.ds(r * HIST + G0, G1)]],
                    rows_v.at[pl.ds(G0, G1)], sem)
                cp0.wait()
                cp1.wait()

                def acc_body(j, carry):
                    a0, a1, a2, a3 = carry
                    a0 = a0 + rows_v[j, pl.ds(0, 16)]
                    a1 = a1 + rows_v[j, pl.ds(16, 16)]
                    a2 = a2 + rows_v[j, pl.ds(32, 16)]
                    a3 = a3 + rows_v[j, pl.ds(48, 16)]
                    return a0, a1, a2, a3

                z = jnp.zeros((16,), jnp.float32)
                a0, a1, a2, a3 = lax.fori_loop(0, HIST, acc_body, (z, z, z, z))
                acc_v[r, pl.ds(0, 16)] = a0
                acc_v[r, pl.ds(16, 16)] = a1
                acc_v[r, pl.ds(32, 16)] = a2
                acc_v[r, pl.ds(48, 16)] = a3
                return 0

            lax.fori_loop(0, CHUNK, row_body, 0)
            pltpu.sync_copy(acc_v, out_hbm.at[pl.ds(row0, CHUNK)])
            return 0

        lax.fori_loop(0, ROWS_PER_W // CHUNK, chunk_body, 0)

    return pool(idx_flat, table)


def _tc_mlp(x, w1, b1, w2, b2):
    TB = 2048

    def body(x_ref, w1_ref, b1_ref, w2_ref, b2_ref, o_ref):
        h = jnp.dot(x_ref[...], w1_ref[...],
                    preferred_element_type=jnp.float32) + b1_ref[...]
        h = h * (1.0 / (1.0 + jnp.exp(-h)))
        o = jnp.dot(h, w2_ref[...], preferred_element_type=jnp.float32) + b2_ref[...]
        o_ref[...] = 1.0 / (1.0 + jnp.exp(-o))

    return pl.pallas_call(
        body,
        grid=(BATCH // TB,),
        in_specs=[
            pl.BlockSpec((TB, EMBED), lambda i: (i, 0)),
            pl.BlockSpec((EMBED, HIDDEN), lambda i: (0, 0)),
            pl.BlockSpec((1, HIDDEN), lambda i: (0, 0)),
            pl.BlockSpec((HIDDEN, OUT), lambda i: (0, 0)),
            pl.BlockSpec((1, OUT), lambda i: (0, 0)),
        ],
        out_specs=pl.BlockSpec((TB, OUT), lambda i: (i, 0)),
        out_shape=jax.ShapeDtypeStruct((BATCH, OUT), jnp.float32),
    )(x, w1, b1, w2, b2)


def kernel(indices, table, W1, b1, W2, b2):
    idx_flat = jnp.reshape(indices, (-1,))
    sums = _sc_pool(idx_flat, table)
    w1s = W1 * (1.0 / HIST)
    return _tc_mlp(sums, w1s, jnp.reshape(b1, (1, HIDDEN)),
                   W2, jnp.reshape(b2, (1, OUT)))


# speedup vs baseline: 3.3607x; 1.7142x over previous
import functools

import jax
import jax.numpy as jnp
from jax import lax
from jax.experimental import pallas as pl
from jax.experimental.pallas import tpu as pltpu
from jax.experimental.pallas import tpu_sc as plsc

VOCAB = 1000000
EMBED = 64
HIDDEN = 256
OUT = 1
BATCH = 16384
HIST = 200

NC = 2
NS = 16
NW = NC * NS
ROWS_PER_W = BATCH // NW
C = 4
NCH = ROWS_PER_W // C
G0 = 128
G1 = HIST - G0
UNROLL = 4


def _sc_pool(idx_flat, table):
    mesh = plsc.VectorSubcoreMesh(core_axis_name="c", subcore_axis_name="s")

    @functools.partial(
        pl.kernel,
        mesh=mesh,
        out_type=jax.ShapeDtypeStruct((BATCH, EMBED), jnp.float32),
        scratch_types=[
            pltpu.VMEM((2, C * HIST), jnp.int32),
            pltpu.VMEM((2, C * HIST, EMBED), jnp.float32),
            pltpu.VMEM((2, C, EMBED), jnp.float32),
            pltpu.SemaphoreType.DMA,
            pltpu.SemaphoreType.DMA,
            pltpu.SemaphoreType.DMA,
            pltpu.SemaphoreType.DMA,
            pltpu.SemaphoreType.DMA,
            pltpu.SemaphoreType.DMA,
        ],
        compiler_params=pltpu.CompilerParams(use_tc_tiling_on_sc=False),
    )
    def pool(idx_hbm, table_hbm, out_hbm, idx_v, rows_v, out_v,
             isem0, isem1, gsem0, gsem1, osem0, osem1):
        wid = lax.axis_index("s") * NC + lax.axis_index("c")
        base = wid * ROWS_PER_W
        isem = (isem0, isem1)
        gsem = (gsem0, gsem1)
        osem = (osem0, osem1)

        def issue_idx(gc, b):
            pltpu.async_copy(
                idx_hbm.at[pl.ds((base + gc * C) * HIST, C * HIST)],
                idx_v.at[b], isem[b])

        def wait_idx(b):
            pltpu.make_async_copy(
                idx_hbm.at[pl.ds(base * HIST, C * HIST)],
                idx_v.at[b], isem[b]).wait()

        def issue_gathers(b):
            for r in range(C):
                pltpu.async_copy(
                    table_hbm.at[idx_v.at[b].at[pl.ds(r * HIST, G0)]],
                    rows_v.at[b].at[pl.ds(r * HIST, G0)], gsem[b])
                pltpu.async_copy(
                    table_hbm.at[idx_v.at[b].at[pl.ds(r * HIST + G0, G1)]],
                    rows_v.at[b].at[pl.ds(r * HIST + G0, G1)], gsem[b])

        def wait_gathers(b):
            pltpu.make_async_copy(
                table_hbm.at[pl.ds(0, C * HIST)], rows_v.at[b],
                gsem[b]).wait()

        def issue_out(gc, b):
            pltpu.async_copy(
                out_v.at[b], out_hbm.at[pl.ds(base + gc * C, C)], osem[b])

        def wait_out(b):
            pltpu.make_async_copy(
                out_v.at[b], out_hbm.at[pl.ds(base, C)], osem[b]).wait()

        for b in (0, 1):
            issue_idx(b, b)
        for b in (0, 1):
            wait_idx(b)
            issue_gathers(b)

        def loop_body(ci2, _):
            for b in (0, 1):
                gc = ci2 * 2 + b
                wait_gathers(b)

                @pl.when(gc + 2 < NCH)
                def _():
                    issue_idx(gc + 2, b)

                for r in range(C):
                    def acc_body(j, carry):
                        a = list(carry)
                        row0 = r * HIST + j * UNROLL
                        for u in range(UNROLL):
                            for c in range(4):
                                k = (u % 2) * 4 + c
                                a[k] = a[k] + rows_v[b, row0 + u,
                                                     pl.ds(c * 16, 16)]
                        return tuple(a)

                    z = jnp.zeros((16,), jnp.float32)
                    accs = lax.fori_loop(0, HIST // UNROLL, acc_body, (z,) * 8)

                    @pl.when(gc >= 2)
                    def _():
                        if r == 0:
                            wait_out(b)

                    for c in range(4):
                        out_v[b, r, pl.ds(c * 16, 16)] = accs[c] + accs[4 + c]

                issue_out(gc, b)

                @pl.when(gc + 2 < NCH)
                def _():
                    wait_idx(b)
                    issue_gathers(b)
            return 0

        lax.fori_loop(0, NCH // 2, loop_body, 0)
        for b in (0, 1):
            wait_out(b)

    return pool(idx_flat, table)


def _tc_mlp(x, w1, b1, w2, b2):
    TB = 2048

    def body(x_ref, w1_ref, b1_ref, w2_ref, b2_ref, o_ref):
        h = jnp.dot(x_ref[...], w1_ref[...],
                    preferred_element_type=jnp.float32) + b1_ref[...]
        h = h * (1.0 / (1.0 + jnp.exp(-h)))
        o = jnp.dot(h, w2_ref[...], preferred_element_type=jnp.float32) + b2_ref[...]
        o_ref[...] = 1.0 / (1.0 + jnp.exp(-o))

    return pl.pallas_call(
        body,
        grid=(BATCH // TB,),
        in_specs=[
            pl.BlockSpec((TB, EMBED), lambda i: (i, 0)),
            pl.BlockSpec((EMBED, HIDDEN), lambda i: (0, 0)),
            pl.BlockSpec((1, HIDDEN), lambda i: (0, 0)),
            pl.BlockSpec((HIDDEN, OUT), lambda i: (0, 0)),
            pl.BlockSpec((1, OUT), lambda i: (0, 0)),
        ],
        out_specs=pl.BlockSpec((TB, OUT), lambda i: (i, 0)),
        out_shape=jax.ShapeDtypeStruct((BATCH, OUT), jnp.float32),
    )(x, w1, b1, w2, b2)


def kernel(indices, table, W1, b1, W2, b2):
    idx_flat = jnp.reshape(indices, (-1,))
    sums = _sc_pool(idx_flat, table)
    w1s = W1 * (1.0 / HIST)
    return _tc_mlp(sums, w1s, jnp.reshape(b1, (1, HIDDEN)),
                   W2, jnp.reshape(b2, (1, OUT)))
